# baseline (speedup 1.0000x reference)
import functools

import jax
import jax.numpy as jnp
from jax import lax
from jax.experimental import pallas as pl
from jax.experimental.pallas import tpu as pltpu
from jax.experimental.pallas import tpu_sc as plsc

_EPS = 1e-09
_N_TASKS = 1000
_N_TASKS_PAD = 1024
_N_SKILLS = 64
_N_SPLITS = 8
_BS = 16384
_D = _N_SKILLS * _N_SPLITS

_NC = 2
_NS = 16
_NW = _NC * _NS
_B_PER_W = _BS // _NW
_T_PER_S = _N_TASKS_PAD // _NS
_TCH = 32
_CH = 32
_N_CHUNKS = _B_PER_W // _CH


def _normalize_rows(buf_v, n_rows):
    lanes = lax.iota(jnp.int32, 16)
    dnums = lax.GatherDimensionNumbers(
        offset_dims=(), collapsed_slice_dims=(0,), start_index_map=(0,)
    )

    def lane_perm(v, idx):
        return lax.gather(
            v,
            idx.reshape(16, 1),
            dnums,
            slice_sizes=(1,),
            mode=lax.GatherScatterMode.PROMISE_IN_BOUNDS,
        )

    def do_row(r, carry):
        for g in range(_N_SPLITS):
            base = g * _N_SKILLS
            vals = []
            for j in range(_N_SKILLS // 16):
                x = buf_v[r, pl.ds(base + j * 16, 16)]
                vals.append(1.0 / (1.0 + jnp.exp(-x)))
            tot = (vals[0] + vals[1]) + (vals[2] + vals[3])
            for k in (8, 4, 2, 1):
                tot = tot + lane_perm(tot, lanes ^ k)
            inv = 1.0 / (tot + _EPS)
            for j in range(_N_SKILLS // 16):
                buf_v[r, pl.ds(base + j * 16, 16)] = vals[j] * inv
        return carry

    lax.fori_loop(0, n_rows, do_row, 0)


def _sc_body(idx_hbm, table_hbm, out_hbm, ptable_hbm, idx_v, gbuf_a, gbuf_b,
             obuf_a, obuf_b, gsem_a, gsem_b, osem_a, osem_b):
    sid = lax.axis_index("s")
    cid = lax.axis_index("c")
    wid = sid * _NC + cid

    trow0 = sid * _T_PER_S
    for t in range(_T_PER_S // _TCH):
        rows = trow0 + t * _TCH
        pltpu.sync_copy(table_hbm.at[pl.ds(rows, _TCH)], gbuf_a)
        _normalize_rows(gbuf_a, _TCH)
        pltpu.sync_copy(gbuf_a, ptable_hbm.at[pl.ds(rows, _TCH)])
    plsc.subcore_barrier()

    base_row = wid * _B_PER_W
    pltpu.sync_copy(idx_hbm.at[pl.ds(base_row, _B_PER_W)], idx_v)

    gbufs = (gbuf_a, gbuf_b)
    obufs = (obuf_a, obuf_b)
    gsems = (gsem_a, gsem_b)
    osems = (osem_a, osem_b)

    def issue_gather(c):
        return pltpu.async_copy(
            ptable_hbm.at[idx_v.at[pl.ds(c * _CH, _CH)]],
            gbufs[c % 2], gsems[c % 2],
        )

    def relayout(gbuf, obuf):
        def do_row(r, carry):
            for g in range(_N_SPLITS):
                for j in range(_N_SKILLS // 16):
                    obuf[r, g, pl.ds(j * 16, 16)] = (
                        gbuf[r, pl.ds(g * _N_SKILLS + j * 16, 16)]
                    )
            return carry

        lax.fori_loop(0, _CH, do_row, 0)

    gather_cps = [None, None]
    out_cps = [None, None]
    gather_cps[0] = issue_gather(0)
    for c in range(_N_CHUNKS):
        b = c % 2
        gather_cps[b].wait()
        if c + 1 < _N_CHUNKS:
            gather_cps[(c + 1) % 2] = issue_gather(c + 1)
        if out_cps[b] is not None:
            out_cps[b].wait()
        relayout(gbufs[b], obufs[b])
        out_cps[b] = pltpu.async_copy(
            obufs[b], out_hbm.at[pl.ds(base_row + c * _CH, _CH)], osems[b]
        )
    out_cps[0].wait()
    out_cps[1].wait()


@functools.partial(
    pl.kernel,
    mesh=plsc.VectorSubcoreMesh(core_axis_name="c", subcore_axis_name="s"),
    out_type=(
        jax.ShapeDtypeStruct((_BS, _N_SPLITS, _N_SKILLS), jnp.float32),
        jax.ShapeDtypeStruct((_N_TASKS_PAD, _D), jnp.float32),
    ),
    scratch_types=[
        pltpu.VMEM((_B_PER_W,), jnp.int32),
        pltpu.VMEM((_CH, _D), jnp.float32),
        pltpu.VMEM((_CH, _D), jnp.float32),
        pltpu.VMEM((_CH, _N_SPLITS, _N_SKILLS), jnp.float32),
        pltpu.VMEM((_CH, _N_SPLITS, _N_SKILLS), jnp.float32),
        pltpu.SemaphoreType.DMA,
        pltpu.SemaphoreType.DMA,
        pltpu.SemaphoreType.DMA,
        pltpu.SemaphoreType.DMA,
    ],
)
def _poly_selector(idx_hbm, table_hbm, out_hbm, ptable_hbm, idx_v, gbuf_a,
                   gbuf_b, obuf_a, obuf_b, gsem_a, gsem_b, osem_a, osem_b):
    _sc_body(idx_hbm, table_hbm, out_hbm, ptable_hbm, idx_v, gbuf_a, gbuf_b,
             obuf_a, obuf_b, gsem_a, gsem_b, osem_a, osem_b)


def kernel(routing_info, weights):
    idx = routing_info.reshape(-1).astype(jnp.int32)
    wpad = jnp.pad(weights, ((0, _N_TASKS_PAD - _N_TASKS), (0, 0)))
    out, _ = _poly_selector(idx, wpad)
    return out
